# baseline (speedup 1.0000x reference)
import functools

import jax
import jax.numpy as jnp
from jax import lax
from jax.experimental import pallas as pl
from jax.experimental.pallas import tpu as pltpu
from jax.experimental.pallas import tpu_sc as plsc

VSZ = 1000000
DSZ = 64
MXLEN = 512
B = 4096
T = 200

_info = plsc.get_sparse_core_info()
_NC, _NS, _L = _info.num_cores, _info.num_subcores, _info.num_lanes
_NW = _NC * _NS
_BBLK = B // _NW
_NBUF = 4


def _body(xt_hbm, w_hbm, pos_hbm, out_hbm, pos_v, ibufs, rows, obufs,
          isems, gsems, wsems):
    wid = lax.axis_index("s") * _NC + lax.axis_index("c")

    ib0 = wid * T * _BBLK
    pltpu.sync_copy(pos_hbm.at[pl.ds(0, T * DSZ)], pos_v)

    iota = lax.iota(jnp.int32, _L)
    zv = iota & 0

    def fire_idx(t, k):
        pltpu.async_copy(xt_hbm.at[pl.ds(ib0 + t * _BBLK, _BBLK)],
                         ibufs[k], isems[k])

    def wait_idx(t, k):
        pltpu.make_async_copy(xt_hbm.at[pl.ds(ib0 + t * _BBLK, _BBLK)],
                              ibufs[k], isems[k]).wait()

    def fire_gather(t, k):
        pltpu.async_copy(w_hbm.at[ibufs[k]], rows[k], gsems[k])

    def wait_gather(t, k):
        pltpu.make_async_copy(w_hbm.at[ibufs[k]], rows[k], gsems[k]).wait()

    def fire_write(t, k):
        pltpu.async_copy(obufs[k],
                         out_hbm.at[pl.ds(8 * t, 8), pl.ds(wid, 1)], wsems[k])

    def wait_write(t, k):
        pltpu.make_async_copy(obufs[k],
                              out_hbm.at[pl.ds(8 * t, 8), pl.ds(wid, 1)],
                              wsems[k]).wait()

    def compute(t, k):
        row = rows[k]
        obuf = obufs[k]

        def d_body(d, c):
            dv = zv + d
            ps = plsc.load_gather(pos_v, [zv + (t * DSZ + d)])
            dt = d >> 3
            for g in range(_BBLK // _L):
                v = plsc.load_gather(row, [iota + _L * g, dv]) + ps
                obuf[dt, 0, d & 7, pl.ds(_L * g, _L)] = v
            return c

        lax.fori_loop(0, DSZ, d_body, 0)

    def b_step(t, k, fire_i, fire_g, wait_w):
        wait_gather(t, k)
        if fire_i:
            fire_idx(t + 4, k)
        if fire_g:
            k2 = (k + 2) % _NBUF
            wait_idx(t + 2, k2)
            fire_gather(t + 2, k2)
        if wait_w:
            wait_write(t - _NBUF, k)
        compute(t, k)
        fire_write(t, k)

    for k in range(_NBUF):
        fire_idx(k, k)
    wait_idx(0, 0)
    fire_gather(0, 0)
    wait_idx(1, 1)
    fire_gather(1, 1)

    for k in range(_NBUF):
        b_step(k, k, True, True, False)

    def outer(g, c):
        t0 = g * _NBUF
        for k in range(_NBUF):
            b_step(t0 + k, k, True, True, True)
        return c

    lax.fori_loop(1, (T // _NBUF) - 1, outer, 0)

    tl = T - _NBUF
    b_step(tl + 0, 0, False, True, True)
    b_step(tl + 1, 1, False, True, True)
    b_step(tl + 2, 2, False, False, True)
    b_step(tl + 3, 3, False, False, True)
    for k in range(_NBUF):
        wait_write(tl + k, k)


def kernel(x, W, pos):
    xt = x.reshape(_NW, _BBLK, T).transpose(0, 2, 1).reshape(-1)
    mesh = plsc.VectorSubcoreMesh(core_axis_name="c", subcore_axis_name="s")
    fn = functools.partial(
        pl.kernel,
        mesh=mesh,
        out_type=jax.ShapeDtypeStruct((T * DSZ // 8, B // _BBLK, 8, _BBLK),
                                      jnp.float32),
        scratch_types=[
            pltpu.VMEM((T * DSZ,), jnp.float32),
            [pltpu.VMEM((_BBLK,), jnp.int32) for _ in range(_NBUF)],
            [pltpu.VMEM((_BBLK, DSZ), jnp.float32) for _ in range(_NBUF)],
            [pltpu.VMEM((DSZ // 8, 1, 8, _BBLK), jnp.float32)
             for _ in range(_NBUF)],
            [pltpu.SemaphoreType.DMA for _ in range(_NBUF)],
            [pltpu.SemaphoreType.DMA for _ in range(_NBUF)],
            [pltpu.SemaphoreType.DMA for _ in range(_NBUF)],
        ],
        compiler_params=pltpu.CompilerParams(use_tc_tiling_on_sc=False,
                                             needs_layout_passes=False),
    )(_body)
    y = fn(xt, W, pos.reshape(-1))
    y5 = y.reshape(T, DSZ // 8, B // _BBLK, 8, _BBLK)
    return y5.transpose(2, 4, 0, 1, 3).reshape(B, T, DSZ)


# speedup vs baseline: 1.4389x; 1.4389x over previous
import functools

import jax
import jax.numpy as jnp
from jax import lax
from jax.experimental import pallas as pl
from jax.experimental.pallas import tpu as pltpu
from jax.experimental.pallas import tpu_sc as plsc

VSZ = 1000000
DSZ = 64
MXLEN = 512
B = 4096
T = 200

_info = plsc.get_sparse_core_info()
_NC, _NS, _L = _info.num_cores, _info.num_subcores, _info.num_lanes
_NW = _NC * _NS
_BBLK = B // _NW
_NBUF = 4


def _body(xt_hbm, w_hbm, pos_hbm, out_hbm, pos_v, ibufs, rows, obufs,
          isems, gsems, wsems):
    wid = lax.axis_index("s") * _NC + lax.axis_index("c")

    ib0 = wid * T * _BBLK
    pltpu.sync_copy(pos_hbm.at[pl.ds(0, T * DSZ)], pos_v)

    iota = lax.iota(jnp.int32, _L)
    zv = iota & 0
    toks = [iota + _L * g for g in range(_BBLK // _L)]

    def fire_idx(t, k):
        pltpu.async_copy(xt_hbm.at[pl.ds(ib0 + t * _BBLK, _BBLK)],
                         ibufs[k], isems[k])

    def wait_idx(t, k):
        pltpu.make_async_copy(xt_hbm.at[pl.ds(ib0 + t * _BBLK, _BBLK)],
                              ibufs[k], isems[k]).wait()

    def fire_gather(t, k):
        pltpu.async_copy(w_hbm.at[ibufs[k]], rows[k], gsems[k])

    def wait_gather(t, k):
        pltpu.make_async_copy(w_hbm.at[ibufs[k]], rows[k], gsems[k]).wait()

    def fire_write(t, k):
        pltpu.async_copy(obufs[k],
                         out_hbm.at[pl.ds(8 * t, 8), pl.ds(wid, 1)], wsems[k])

    def wait_write(t, k):
        pltpu.make_async_copy(obufs[k],
                              out_hbm.at[pl.ds(8 * t, 8), pl.ds(wid, 1)],
                              wsems[k]).wait()

    def compute(t, k):
        row = rows[k]
        obuf = obufs[k]

        @plsc.parallel_loop(0, DSZ, 1, unroll=8)
        def d_body(d):
            dv = zv + d
            ps = plsc.load_gather(pos_v, [zv + (t * DSZ + d)])
            dt = d >> 3
            for g in range(_BBLK // _L):
                v = plsc.load_gather(row, [toks[g], dv]) + ps
                obuf[dt, 0, d & 7, pl.ds(_L * g, _L)] = v

    def b_step(t, k, fire_i, fire_g, wait_w):
        wait_gather(t, k)
        if fire_i:
            fire_idx(t + 4, k)
        if fire_g:
            k2 = (k + 2) % _NBUF
            wait_idx(t + 2, k2)
            fire_gather(t + 2, k2)
        if wait_w:
            wait_write(t - _NBUF, k)
        compute(t, k)
        fire_write(t, k)

    for k in range(_NBUF):
        fire_idx(k, k)
    wait_idx(0, 0)
    fire_gather(0, 0)
    wait_idx(1, 1)
    fire_gather(1, 1)

    for k in range(_NBUF):
        b_step(k, k, True, True, False)

    def outer(g, c):
        t0 = g * _NBUF
        for k in range(_NBUF):
            b_step(t0 + k, k, True, True, True)
        return c

    lax.fori_loop(1, (T // _NBUF) - 1, outer, 0)

    tl = T - _NBUF
    b_step(tl + 0, 0, False, True, True)
    b_step(tl + 1, 1, False, True, True)
    b_step(tl + 2, 2, False, False, True)
    b_step(tl + 3, 3, False, False, True)
    for k in range(_NBUF):
        wait_write(tl + k, k)


def kernel(x, W, pos):
    xt = x.reshape(_NW, _BBLK, T).transpose(0, 2, 1).reshape(-1)
    mesh = plsc.VectorSubcoreMesh(core_axis_name="c", subcore_axis_name="s")
    fn = functools.partial(
        pl.kernel,
        mesh=mesh,
        out_type=jax.ShapeDtypeStruct((T * DSZ // 8, B // _BBLK, 8, _BBLK),
                                      jnp.float32),
        scratch_types=[
            pltpu.VMEM((T * DSZ,), jnp.float32),
            [pltpu.VMEM((_BBLK,), jnp.int32) for _ in range(_NBUF)],
            [pltpu.VMEM((_BBLK, DSZ), jnp.float32) for _ in range(_NBUF)],
            [pltpu.VMEM((DSZ // 8, 1, 8, _BBLK), jnp.float32)
             for _ in range(_NBUF)],
            [pltpu.SemaphoreType.DMA for _ in range(_NBUF)],
            [pltpu.SemaphoreType.DMA for _ in range(_NBUF)],
            [pltpu.SemaphoreType.DMA for _ in range(_NBUF)],
        ],
        compiler_params=pltpu.CompilerParams(use_tc_tiling_on_sc=False,
                                             needs_layout_passes=False),
    )(_body)
    y = fn(xt, W, pos.reshape(-1))
    y5 = y.reshape(T, DSZ // 8, B // _BBLK, 8, _BBLK)
    return y5.transpose(2, 4, 0, 1, 3).reshape(B, T, DSZ)


# speedup vs baseline: 2.5569x; 1.7770x over previous
import functools

import jax
import jax.numpy as jnp
from jax import lax
from jax.experimental import pallas as pl
from jax.experimental.pallas import tpu as pltpu
from jax.experimental.pallas import tpu_sc as plsc

VSZ = 1000000
DSZ = 64
MXLEN = 512
B = 4096
T = 200

_info = plsc.get_sparse_core_info()
_NC, _NS, _L = _info.num_cores, _info.num_subcores, _info.num_lanes
_NW = _NC * _NS
_BBLK = B // _NW
_NBUF = 4


def _body(xt_hbm, w_hbm, pos_hbm, out_hbm, pos_v, ibufs, rows, obufs,
          isems, gsems, wsems):
    wid = lax.axis_index("s") * _NC + lax.axis_index("c")

    ib0 = wid * T * _BBLK
    pltpu.sync_copy(pos_hbm.at[pl.ds(0, T * DSZ)], pos_v)

    iota = lax.iota(jnp.int32, _L)
    zv = iota & 0
    dlane = [iota + _L * j for j in range(DSZ // _L)]
    base = [((d >> 3) * (8 * 129) + (d & 7) * 129) for d in dlane]

    def fire_idx(t, k):
        pltpu.async_copy(xt_hbm.at[pl.ds(ib0 + t * _BBLK, _BBLK)],
                         ibufs[k], isems[k])

    def wait_idx(t, k):
        pltpu.make_async_copy(xt_hbm.at[pl.ds(ib0 + t * _BBLK, _BBLK)],
                              ibufs[k], isems[k]).wait()

    def fire_gather(t, k):
        pltpu.async_copy(w_hbm.at[ibufs[k]], rows[k], gsems[k])

    def wait_gather(t, k):
        pltpu.make_async_copy(w_hbm.at[ibufs[k]], rows[k], gsems[k]).wait()

    def fire_write(t, k):
        pltpu.async_copy(obufs[k].at[:, :, :, pl.ds(0, _BBLK)],
                         out_hbm.at[pl.ds(8 * t, 8), pl.ds(wid, 1)], wsems[k])

    def wait_write(t, k):
        pltpu.make_async_copy(obufs[k].at[:, :, :, pl.ds(0, _BBLK)],
                              out_hbm.at[pl.ds(8 * t, 8), pl.ds(wid, 1)],
                              wsems[k]).wait()

    def compute(t, k):
        row = rows[k]
        obuf = obufs[k]
        pk = [pos_v[pl.ds(t * DSZ + _L * j, _L)] for j in range(DSZ // _L)]

        @plsc.parallel_loop(0, _BBLK, 1, unroll=4)
        def tok_body(tok):
            tv = zv + tok
            for j in range(DSZ // _L):
                v = row[tok, pl.ds(_L * j, _L)] + pk[j]
                plsc.store_scatter(obuf, [zv, zv, zv, base[j] + tv], v)

    def b_step(t, k, fire_i, fire_g, wait_w):
        wait_gather(t, k)
        if fire_i:
            fire_idx(t + 4, k)
        if fire_g:
            k2 = (k + 2) % _NBUF
            wait_idx(t + 2, k2)
            fire_gather(t + 2, k2)
        if wait_w:
            wait_write(t - _NBUF, k)
        compute(t, k)
        fire_write(t, k)

    for k in range(_NBUF):
        fire_idx(k, k)
    wait_idx(0, 0)
    fire_gather(0, 0)
    wait_idx(1, 1)
    fire_gather(1, 1)

    for k in range(_NBUF):
        b_step(k, k, True, True, False)

    def outer(g, c):
        t0 = g * _NBUF
        for k in range(_NBUF):
            b_step(t0 + k, k, True, True, True)
        return c

    lax.fori_loop(1, (T // _NBUF) - 1, outer, 0)

    tl = T - _NBUF
    b_step(tl + 0, 0, False, True, True)
    b_step(tl + 1, 1, False, True, True)
    b_step(tl + 2, 2, False, False, True)
    b_step(tl + 3, 3, False, False, True)
    for k in range(_NBUF):
        wait_write(tl + k, k)


def kernel(x, W, pos):
    xt = x.reshape(_NW, _BBLK, T).transpose(0, 2, 1).reshape(-1)
    mesh = plsc.VectorSubcoreMesh(core_axis_name="c", subcore_axis_name="s")
    fn = functools.partial(
        pl.kernel,
        mesh=mesh,
        out_type=jax.ShapeDtypeStruct((T * DSZ // 8, B // _BBLK, 8, _BBLK),
                                      jnp.float32),
        scratch_types=[
            pltpu.VMEM((T * DSZ,), jnp.float32),
            [pltpu.VMEM((_BBLK,), jnp.int32) for _ in range(_NBUF)],
            [pltpu.VMEM((_BBLK, DSZ), jnp.float32) for _ in range(_NBUF)],
            [pltpu.VMEM((DSZ // 8, 1, 8, _BBLK + 1), jnp.float32)
             for _ in range(_NBUF)],
            [pltpu.SemaphoreType.DMA for _ in range(_NBUF)],
            [pltpu.SemaphoreType.DMA for _ in range(_NBUF)],
            [pltpu.SemaphoreType.DMA for _ in range(_NBUF)],
        ],
        compiler_params=pltpu.CompilerParams(use_tc_tiling_on_sc=False,
                                             needs_layout_passes=False),
    )(_body)
    y = fn(xt, W, pos.reshape(-1))
    y5 = y.reshape(T, DSZ // 8, B // _BBLK, 8, _BBLK)
    return y5.transpose(2, 4, 0, 1, 3).reshape(B, T, DSZ)
